# Initial kernel scaffold; baseline (speedup 1.0000x reference)
import jax
import jax.numpy as jnp
from jax.experimental import pallas as pl


def kernel(x, embedding):
    raise NotImplementedError("write your pallas kernel here")


# speedup vs baseline: 1.2267x; 1.2267x over previous
import functools

import jax
import jax.numpy as jnp
from jax import lax
from jax.experimental import pallas as pl
from jax.experimental.pallas import tpu as pltpu
from jax.experimental.pallas import tpu_sc as plsc

K_CODES = 8192
D = 256
NB = 8
HW = 1024
N_TOK = NB * HW
KB = 2048
NKB = K_CODES // KB

def _assign_body(x_ref, e_ref, idx_ref, bestd_ref):
    k = pl.program_id(0)
    b = pl.program_id(1)
    x = x_ref[0]
    e = e_ref[...]
    mm = lax.dot_general(e.astype(jnp.bfloat16), x.astype(jnp.bfloat16),
                         (((1,), (0,)), ((), ())),
                         preferred_element_type=jnp.float32)
    en = jnp.sum(e * e, axis=1, keepdims=True)
    xn = jnp.sum(x * x, axis=0, keepdims=True)
    d2 = jnp.maximum(en + xn - 2.0 * mm, 0.0)
    dist = jnp.sqrt(d2)
    lmin = jnp.min(dist, axis=0, keepdims=True)
    rows = lax.broadcasted_iota(jnp.int32, (KB, HW), 0)
    larg = jnp.min(jnp.where(dist == lmin, rows, K_CODES),
                   axis=0, keepdims=True) + k * KB

    @pl.when(k == 0)
    def _init():
        bestd_ref[pl.ds(b, 1), :] = lmin
        idx_ref[pl.ds(b, 1), :] = larg

    @pl.when(k > 0)
    def _update():
        prev_d = bestd_ref[pl.ds(b, 1), :]
        prev_i = idx_ref[pl.ds(b, 1), :]
        better = lmin < prev_d
        bestd_ref[pl.ds(b, 1), :] = jnp.where(better, lmin, prev_d)
        idx_ref[pl.ds(b, 1), :] = jnp.where(better, larg, prev_i)


def _assign(x3, embedding):
    return pl.pallas_call(
        _assign_body,
        grid=(NKB, NB),
        in_specs=[
            pl.BlockSpec((1, D, HW), lambda k, b: (b, 0, 0)),
            pl.BlockSpec((KB, D), lambda k, b: (k, 0)),
        ],
        out_specs=pl.BlockSpec((NB, HW), lambda k, b: (0, 0)),
        out_shape=jax.ShapeDtypeStruct((NB, HW), jnp.int32),
        scratch_shapes=[pltpu.VMEM((NB, HW), jnp.float32)],
        compiler_params=pltpu.CompilerParams(
            dimension_semantics=("arbitrary", "arbitrary")),
    )(x3, embedding)


NC = 2
NS = 16
NW = NC * NS
ROWS_PER_W = N_TOK // NW
IDX_ROWS_PER_W = ROWS_PER_W // 128


def _sc_gather_counts(idx64, embedding):
    mesh = plsc.VectorSubcoreMesh(core_axis_name="c", subcore_axis_name="s")

    @functools.partial(
        pl.kernel,
        mesh=mesh,
        out_type=[
            jax.ShapeDtypeStruct((N_TOK, D), jnp.float32),
            jax.ShapeDtypeStruct((NC, K_CODES), jnp.float32),
        ],
        scratch_types=[
            pltpu.VMEM((IDX_ROWS_PER_W, 128), jnp.int32),
            pltpu.VMEM((ROWS_PER_W, D), jnp.float32),
            pltpu.VMEM((128,), jnp.float32),
            pltpu.VMEM((K_CODES // NS,), jnp.float32),
            pltpu.VMEM_SHARED((K_CODES,), jnp.float32),
            pltpu.SemaphoreType.DMA,
        ],
    )
    def sc_kernel(idx_hbm, table_hbm, qout_hbm, counts_hbm,
                  idx_v, rows_v, ones_v, zeros_v, counts_sh, sem):
        cid = lax.axis_index("c")
        sid = lax.axis_index("s")
        wid = cid * NS + sid

        pltpu.sync_copy(idx_hbm.at[pl.ds(wid * IDX_ROWS_PER_W, IDX_ROWS_PER_W)],
                        idx_v)
        cp0 = pltpu.async_copy(table_hbm.at[idx_v.at[0]],
                               rows_v.at[pl.ds(0, 128)], sem)
        cp1 = pltpu.async_copy(table_hbm.at[idx_v.at[1]],
                               rows_v.at[pl.ds(128, 128)], sem)

        def _z(i, carry):
            zeros_v[pl.ds(i * 16, 16)] = jnp.zeros((16,), jnp.float32)
            return carry
        lax.fori_loop(0, (K_CODES // NS) // 16, _z, 0)
        pltpu.sync_copy(zeros_v,
                        counts_sh.at[pl.ds(sid * (K_CODES // NS),
                                           K_CODES // NS)])
        for j in range(128 // 16):
            ones_v[pl.ds(j * 16, 16)] = jnp.ones((16,), jnp.float32)
        plsc.subcore_barrier()

        pltpu.sync_copy(ones_v, counts_sh.at[idx_v.at[0]], add=True)
        pltpu.sync_copy(ones_v, counts_sh.at[idx_v.at[1]], add=True)
        plsc.subcore_barrier()

        @pl.when(sid == 0)
        def _dump_counts():
            pltpu.sync_copy(counts_sh, counts_hbm.at[cid])

        cp0.wait()
        cp1.wait()
        pltpu.sync_copy(rows_v, qout_hbm.at[pl.ds(wid * ROWS_PER_W,
                                                  ROWS_PER_W)])

    return sc_kernel(idx64, embedding)


def _finish_body(x_ref, q_ref, c_ref, qst_ref, loss_ref, perp_ref):
    b = pl.program_id(0)
    x = x_ref[0]
    q = q_ref[...]
    qt = jnp.transpose(q, (1, 0))
    qst_ref[0] = x + (qt - x)
    diff = x - qt
    part = jnp.sum(diff * diff)

    @pl.when(b == 0)
    def _first():
        loss_ref[0, 0] = part
        csum = c_ref[0] + c_ref[1]
        p = csum * (1.0 / N_TOK)
        ent = jnp.sum(p * jnp.log(p + 1e-10))
        perp_ref[0, 0] = jnp.exp(-ent)

    @pl.when(b > 0)
    def _acc():
        loss_ref[0, 0] += part

    @pl.when(b == NB - 1)
    def _final():
        loss_ref[0, 0] = loss_ref[0, 0] * (0.25 / (N_TOK * D))


def _finish(x3, qflat, counts3):
    return pl.pallas_call(
        _finish_body,
        grid=(NB,),
        in_specs=[
            pl.BlockSpec((1, D, HW), lambda b: (b, 0, 0)),
            pl.BlockSpec((HW, D), lambda b: (b, 0)),
            pl.BlockSpec((NC, 64, 128), lambda b: (0, 0, 0)),
        ],
        out_specs=[
            pl.BlockSpec((1, D, HW), lambda b: (b, 0, 0)),
            pl.BlockSpec(memory_space=pltpu.SMEM),
            pl.BlockSpec(memory_space=pltpu.SMEM),
        ],
        out_shape=[
            jax.ShapeDtypeStruct((NB, D, HW), jnp.float32),
            jax.ShapeDtypeStruct((1, 1), jnp.float32),
            jax.ShapeDtypeStruct((1, 1), jnp.float32),
        ],
        compiler_params=pltpu.CompilerParams(
            dimension_semantics=("arbitrary",)),
    )(x3, qflat, counts3)


def kernel(x, embedding):
    x3 = x.reshape(NB, D, HW)
    idx = _assign(x3, embedding)
    idx64 = idx.reshape(N_TOK // 128, 128)
    qflat, counts = _sc_gather_counts(idx64, embedding)
    counts3 = counts.reshape(NC, 64, 128)
    qst3, loss, perp = _finish(x3, qflat, counts3)
    return (qst3.reshape(NB, D, 32, 32), loss[0, 0], perp[0, 0])


# speedup vs baseline: 1.7349x; 1.4143x over previous
import functools

import jax
import jax.numpy as jnp
from jax import lax
from jax.experimental import pallas as pl
from jax.experimental.pallas import tpu as pltpu
from jax.experimental.pallas import tpu_sc as plsc

K_CODES = 8192
D = 256
NB = 8
HW = 1024
N_TOK = NB * HW
KB = 2048
NKB = K_CODES // KB

def _assign_body(x_ref, e_ref, idx_ref, bestd_ref):
    k = pl.program_id(0)
    b = pl.program_id(1)
    x = x_ref[0]
    e = e_ref[...]
    eb2 = e.astype(jnp.bfloat16) * jnp.asarray(2.0, jnp.bfloat16)
    h = lax.dot_general(eb2, x.astype(jnp.bfloat16),
                        (((1,), (0,)), ((), ())),
                        preferred_element_type=jnp.float32)
    en = jnp.sum(e * e, axis=1, keepdims=True)
    xn = jnp.sum(x * x, axis=0, keepdims=True)
    d2 = (en + xn) - h
    mraw = jnp.min(d2, axis=0, keepdims=True)
    m = jnp.maximum(mraw, 0.0)
    lmin = jnp.sqrt(m)
    sn = lax.bitcast_convert_type(
        lax.bitcast_convert_type(lmin, jnp.int32) + 1, jnp.float32)
    t = sn * sn
    for _ in range(4):
        tdn = lax.bitcast_convert_type(
            lax.bitcast_convert_type(t, jnp.int32) - 1, jnp.float32)
        t = jnp.where(jnp.sqrt(t) >= sn, tdn, t)
    for _ in range(4):
        tup = lax.bitcast_convert_type(
            lax.bitcast_convert_type(t, jnp.int32) + 1, jnp.float32)
        t = jnp.where(jnp.sqrt(tup) < sn, tup, t)
    t = jnp.where(m == 0.0, 0.0, jnp.maximum(t, mraw))
    rows = lax.broadcasted_iota(jnp.int32, (KB, HW), 0).astype(jnp.float32)
    larg = jnp.min(jnp.where(d2 <= t, rows, float(K_CODES)),
                   axis=0, keepdims=True).astype(jnp.int32) + k * KB

    @pl.when(k == 0)
    def _init():
        bestd_ref[pl.ds(b, 1), :] = lmin
        idx_ref[pl.ds(b, 1), :] = larg

    @pl.when(k > 0)
    def _update():
        prev_d = bestd_ref[pl.ds(b, 1), :]
        prev_i = idx_ref[pl.ds(b, 1), :]
        better = lmin < prev_d
        bestd_ref[pl.ds(b, 1), :] = jnp.where(better, lmin, prev_d)
        idx_ref[pl.ds(b, 1), :] = jnp.where(better, larg, prev_i)


def _assign(x3, embedding):
    return pl.pallas_call(
        _assign_body,
        grid=(NKB, NB),
        in_specs=[
            pl.BlockSpec((1, D, HW), lambda k, b: (b, 0, 0)),
            pl.BlockSpec((KB, D), lambda k, b: (k, 0)),
        ],
        out_specs=pl.BlockSpec((NB, HW), lambda k, b: (0, 0)),
        out_shape=jax.ShapeDtypeStruct((NB, HW), jnp.int32),
        scratch_shapes=[pltpu.VMEM((NB, HW), jnp.float32)],
        compiler_params=pltpu.CompilerParams(
            dimension_semantics=("arbitrary", "arbitrary")),
    )(x3, embedding)


NC = 2
NS = 16
NW = NC * NS
ROWS_PER_W = N_TOK // NW
IDX_ROWS_PER_W = ROWS_PER_W // 128


def _sc_gather_counts(idx64, embedding):
    mesh = plsc.VectorSubcoreMesh(core_axis_name="c", subcore_axis_name="s")

    @functools.partial(
        pl.kernel,
        mesh=mesh,
        out_type=[
            jax.ShapeDtypeStruct((N_TOK, D), jnp.float32),
            jax.ShapeDtypeStruct((NC, K_CODES), jnp.float32),
        ],
        scratch_types=[
            pltpu.VMEM((IDX_ROWS_PER_W, 128), jnp.int32),
            pltpu.VMEM((ROWS_PER_W, D), jnp.float32),
            pltpu.VMEM((128,), jnp.float32),
            pltpu.VMEM((K_CODES // NS,), jnp.float32),
            pltpu.VMEM_SHARED((K_CODES,), jnp.float32),
            pltpu.SemaphoreType.DMA,
        ],
    )
    def sc_kernel(idx_hbm, table_hbm, qout_hbm, counts_hbm,
                  idx_v, rows_v, ones_v, zeros_v, counts_sh, sem):
        cid = lax.axis_index("c")
        sid = lax.axis_index("s")
        wid = cid * NS + sid

        pltpu.sync_copy(idx_hbm.at[pl.ds(wid * IDX_ROWS_PER_W, IDX_ROWS_PER_W)],
                        idx_v)
        cp0 = pltpu.async_copy(table_hbm.at[idx_v.at[0]],
                               rows_v.at[pl.ds(0, 128)], sem)
        cp1 = pltpu.async_copy(table_hbm.at[idx_v.at[1]],
                               rows_v.at[pl.ds(128, 128)], sem)

        def _z(i, carry):
            zeros_v[pl.ds(i * 16, 16)] = jnp.zeros((16,), jnp.float32)
            return carry
        lax.fori_loop(0, (K_CODES // NS) // 16, _z, 0)
        pltpu.sync_copy(zeros_v,
                        counts_sh.at[pl.ds(sid * (K_CODES // NS),
                                           K_CODES // NS)])
        for j in range(128 // 16):
            ones_v[pl.ds(j * 16, 16)] = jnp.ones((16,), jnp.float32)
        plsc.subcore_barrier()

        pltpu.sync_copy(ones_v, counts_sh.at[idx_v.at[0]], add=True)
        pltpu.sync_copy(ones_v, counts_sh.at[idx_v.at[1]], add=True)
        plsc.subcore_barrier()

        @pl.when(sid == 0)
        def _dump_counts():
            pltpu.sync_copy(counts_sh, counts_hbm.at[cid])

        cp0.wait()
        cp1.wait()
        pltpu.sync_copy(rows_v, qout_hbm.at[pl.ds(wid * ROWS_PER_W,
                                                  ROWS_PER_W)])

    return sc_kernel(idx64, embedding)


def _finish_body(x_ref, q_ref, c_ref, qst_ref, loss_ref, perp_ref):
    b = pl.program_id(0)
    x = x_ref[0]
    q = q_ref[...]
    qt = jnp.transpose(q, (1, 0))
    qst_ref[0] = x + (qt - x)
    diff = x - qt
    part = jnp.sum(diff * diff)

    @pl.when(b == 0)
    def _first():
        loss_ref[0, 0] = part
        csum = c_ref[0] + c_ref[1]
        p = csum * (1.0 / N_TOK)
        ent = jnp.sum(p * jnp.log(p + 1e-10))
        perp_ref[0, 0] = jnp.exp(-ent)

    @pl.when(b > 0)
    def _acc():
        loss_ref[0, 0] += part

    @pl.when(b == NB - 1)
    def _final():
        loss_ref[0, 0] = loss_ref[0, 0] * (0.25 / (N_TOK * D))


def _finish(x3, qflat, counts3):
    return pl.pallas_call(
        _finish_body,
        grid=(NB,),
        in_specs=[
            pl.BlockSpec((1, D, HW), lambda b: (b, 0, 0)),
            pl.BlockSpec((HW, D), lambda b: (b, 0)),
            pl.BlockSpec((NC, 64, 128), lambda b: (0, 0, 0)),
        ],
        out_specs=[
            pl.BlockSpec((1, D, HW), lambda b: (b, 0, 0)),
            pl.BlockSpec(memory_space=pltpu.SMEM),
            pl.BlockSpec(memory_space=pltpu.SMEM),
        ],
        out_shape=[
            jax.ShapeDtypeStruct((NB, D, HW), jnp.float32),
            jax.ShapeDtypeStruct((1, 1), jnp.float32),
            jax.ShapeDtypeStruct((1, 1), jnp.float32),
        ],
        compiler_params=pltpu.CompilerParams(
            dimension_semantics=("arbitrary",)),
    )(x3, qflat, counts3)


def kernel(x, embedding):
    x3 = x.reshape(NB, D, HW)
    idx = _assign(x3, embedding)
    idx64 = idx.reshape(N_TOK // 128, 128)
    qflat, counts = _sc_gather_counts(idx64, embedding)
    counts3 = counts.reshape(NC, 64, 128)
    qst3, loss, perp = _finish(x3, qflat, counts3)
    return (qst3.reshape(NB, D, 32, 32), loss[0, 0], perp[0, 0])


# speedup vs baseline: 1.8090x; 1.0427x over previous
import functools

import jax
import jax.numpy as jnp
from jax import lax
from jax.experimental import pallas as pl
from jax.experimental.pallas import tpu as pltpu
from jax.experimental.pallas import tpu_sc as plsc

K_CODES = 8192
D = 256
NB = 8
HW = 1024
N_TOK = NB * HW
KB = 4096
NKB = K_CODES // KB

def _assign_body(x_ref, e_ref, idx_ref, bestd_ref):
    k = pl.program_id(0)
    b = pl.program_id(1)
    x = x_ref[0]
    e = e_ref[...]
    eb2 = e.astype(jnp.bfloat16) * jnp.asarray(2.0, jnp.bfloat16)
    h = lax.dot_general(eb2, x.astype(jnp.bfloat16),
                        (((1,), (0,)), ((), ())),
                        preferred_element_type=jnp.float32)
    en = jnp.sum(e * e, axis=1, keepdims=True)
    xn = jnp.sum(x * x, axis=0, keepdims=True)
    d2 = (en + xn) - h
    mraw = jnp.min(d2, axis=0, keepdims=True)
    m = jnp.maximum(mraw, 0.0)
    lmin = jnp.sqrt(m)
    sn = lax.bitcast_convert_type(
        lax.bitcast_convert_type(lmin, jnp.int32) + 1, jnp.float32)
    t = sn * sn
    for _ in range(4):
        tdn = lax.bitcast_convert_type(
            lax.bitcast_convert_type(t, jnp.int32) - 1, jnp.float32)
        t = jnp.where(jnp.sqrt(t) >= sn, tdn, t)
    for _ in range(4):
        tup = lax.bitcast_convert_type(
            lax.bitcast_convert_type(t, jnp.int32) + 1, jnp.float32)
        t = jnp.where(jnp.sqrt(tup) < sn, tup, t)
    t = jnp.where(m == 0.0, 0.0, jnp.maximum(t, mraw))
    rows = lax.broadcasted_iota(jnp.int32, (KB, HW), 0).astype(jnp.float32)
    larg = jnp.min(jnp.where(d2 <= t, rows, float(K_CODES)),
                   axis=0, keepdims=True).astype(jnp.int32) + k * KB

    @pl.when(k == 0)
    def _init():
        bestd_ref[pl.ds(b, 1), :] = lmin
        idx_ref[pl.ds(b, 1), :] = larg

    @pl.when(k > 0)
    def _update():
        prev_d = bestd_ref[pl.ds(b, 1), :]
        prev_i = idx_ref[pl.ds(b, 1), :]
        better = lmin < prev_d
        bestd_ref[pl.ds(b, 1), :] = jnp.where(better, lmin, prev_d)
        idx_ref[pl.ds(b, 1), :] = jnp.where(better, larg, prev_i)


def _assign(x3, embedding):
    return pl.pallas_call(
        _assign_body,
        grid=(NKB, NB),
        in_specs=[
            pl.BlockSpec((1, D, HW), lambda k, b: (b, 0, 0)),
            pl.BlockSpec((KB, D), lambda k, b: (k, 0)),
        ],
        out_specs=pl.BlockSpec((NB, HW), lambda k, b: (0, 0)),
        out_shape=jax.ShapeDtypeStruct((NB, HW), jnp.int32),
        scratch_shapes=[pltpu.VMEM((NB, HW), jnp.float32)],
        compiler_params=pltpu.CompilerParams(
            dimension_semantics=("arbitrary", "arbitrary")),
    )(x3, embedding)


NC = 2
NS = 16
NW = NC * NS
ROWS_PER_W = N_TOK // NW
IDX_ROWS_PER_W = ROWS_PER_W // 128


def _sc_gather_counts(idx64, embedding):
    mesh = plsc.VectorSubcoreMesh(core_axis_name="c", subcore_axis_name="s")

    @functools.partial(
        pl.kernel,
        mesh=mesh,
        out_type=[
            jax.ShapeDtypeStruct((N_TOK, D), jnp.float32),
            jax.ShapeDtypeStruct((NC, K_CODES), jnp.float32),
        ],
        scratch_types=[
            pltpu.VMEM((IDX_ROWS_PER_W, 128), jnp.int32),
            pltpu.VMEM((ROWS_PER_W, D), jnp.float32),
            pltpu.VMEM((128,), jnp.float32),
            pltpu.VMEM((K_CODES // NS,), jnp.float32),
            pltpu.VMEM_SHARED((K_CODES,), jnp.float32),
            pltpu.SemaphoreType.DMA,
        ],
    )
    def sc_kernel(idx_hbm, table_hbm, qout_hbm, counts_hbm,
                  idx_v, rows_v, ones_v, zeros_v, counts_sh, sem):
        cid = lax.axis_index("c")
        sid = lax.axis_index("s")
        wid = cid * NS + sid

        pltpu.sync_copy(idx_hbm.at[pl.ds(wid * IDX_ROWS_PER_W, IDX_ROWS_PER_W)],
                        idx_v)
        cp0 = pltpu.async_copy(table_hbm.at[idx_v.at[0]],
                               rows_v.at[pl.ds(0, 128)], sem)
        cp1 = pltpu.async_copy(table_hbm.at[idx_v.at[1]],
                               rows_v.at[pl.ds(128, 128)], sem)

        def _z(i, carry):
            zeros_v[pl.ds(i * 16, 16)] = jnp.zeros((16,), jnp.float32)
            return carry
        lax.fori_loop(0, (K_CODES // NS) // 16, _z, 0)
        pltpu.sync_copy(zeros_v,
                        counts_sh.at[pl.ds(sid * (K_CODES // NS),
                                           K_CODES // NS)])
        for j in range(128 // 16):
            ones_v[pl.ds(j * 16, 16)] = jnp.ones((16,), jnp.float32)
        plsc.subcore_barrier()

        pltpu.sync_copy(ones_v, counts_sh.at[idx_v.at[0]], add=True)
        pltpu.sync_copy(ones_v, counts_sh.at[idx_v.at[1]], add=True)
        plsc.subcore_barrier()

        @pl.when(sid == 0)
        def _dump_counts():
            pltpu.sync_copy(counts_sh, counts_hbm.at[cid])

        cp0.wait()
        cp1.wait()
        pltpu.sync_copy(rows_v, qout_hbm.at[pl.ds(wid * ROWS_PER_W,
                                                  ROWS_PER_W)])

    return sc_kernel(idx64, embedding)


def _finish_body(x_ref, q_ref, c_ref, qst_ref, loss_ref, perp_ref):
    b = pl.program_id(0)
    x = x_ref[0]
    q = q_ref[...]
    qt = jnp.transpose(q, (1, 0))
    qst_ref[0] = x + (qt - x)
    diff = x - qt
    part = jnp.sum(diff * diff)

    @pl.when(b == 0)
    def _first():
        loss_ref[0, 0] = part
        csum = c_ref[0] + c_ref[1]
        p = csum * (1.0 / N_TOK)
        ent = jnp.sum(p * jnp.log(p + 1e-10))
        perp_ref[0, 0] = jnp.exp(-ent)

    @pl.when(b > 0)
    def _acc():
        loss_ref[0, 0] += part

    @pl.when(b == NB - 1)
    def _final():
        loss_ref[0, 0] = loss_ref[0, 0] * (0.25 / (N_TOK * D))


def _finish(x3, qflat, counts3):
    return pl.pallas_call(
        _finish_body,
        grid=(NB,),
        in_specs=[
            pl.BlockSpec((1, D, HW), lambda b: (b, 0, 0)),
            pl.BlockSpec((HW, D), lambda b: (b, 0)),
            pl.BlockSpec((NC, 64, 128), lambda b: (0, 0, 0)),
        ],
        out_specs=[
            pl.BlockSpec((1, D, HW), lambda b: (b, 0, 0)),
            pl.BlockSpec(memory_space=pltpu.SMEM),
            pl.BlockSpec(memory_space=pltpu.SMEM),
        ],
        out_shape=[
            jax.ShapeDtypeStruct((NB, D, HW), jnp.float32),
            jax.ShapeDtypeStruct((1, 1), jnp.float32),
            jax.ShapeDtypeStruct((1, 1), jnp.float32),
        ],
        compiler_params=pltpu.CompilerParams(
            dimension_semantics=("arbitrary",)),
    )(x3, qflat, counts3)


def kernel(x, embedding):
    x3 = x.reshape(NB, D, HW)
    idx = _assign(x3, embedding)
    idx64 = idx.reshape(N_TOK // 128, 128)
    qflat, counts = _sc_gather_counts(idx64, embedding)
    counts3 = counts.reshape(NC, 64, 128)
    qst3, loss, perp = _finish(x3, qflat, counts3)
    return (qst3.reshape(NB, D, 32, 32), loss[0, 0], perp[0, 0])
